# baseline (speedup 1.0000x reference)
import functools

import jax
import jax.numpy as jnp
from jax import lax
from jax.experimental import pallas as pl
from jax.experimental.pallas import tpu as pltpu
from jax.experimental.pallas import tpu_sc as plsc

ENT_N = 1000000
REL_N = 1000
DIM = 32
B = 16384
PKD = DIM // 2

NC = 2
NS = 16
NW = NC * NS
BPW = B // NW
CHUNK = 128
NCHUNK = BPW // CHUNK
GPC = CHUNK // 16


def _tec_body(h_hbm, r_hbm, t_hbm, entp_hbm, rel1_hbm, out_hbm,
              h_sm, r_sm, t_sm, rel_v,
              h_b0, t_b0, h_b1, t_b1, scores, sem0, sem1):
    wid = lax.axis_index("s") * NC + lax.axis_index("c")

    pltpu.sync_copy(h_hbm.at[pl.ds(wid * BPW, BPW)], h_sm)
    pltpu.sync_copy(r_hbm.at[pl.ds(wid * BPW, BPW)], r_sm)
    pltpu.sync_copy(t_hbm.at[pl.ds(wid * BPW, BPW)], t_sm)
    pltpu.sync_copy(rel1_hbm, rel_v)

    bufs = [(h_b0, t_b0, sem0), (h_b1, t_b1, sem1)]
    lane = lax.iota(jnp.int32, 16)
    himask = jnp.full((16,), jnp.int32(-65536))

    def fire(c):
        hbuf, tbuf, sem = bufs[c % 2]

        def fire_g(g, carry):
            base = c * CHUNK + g * 16
            hvec = h_sm[pl.ds(base, 16)]
            tvec = t_sm[pl.ds(base, 16)]
            for k in range(16):
                b = g * 16 + k
                pltpu.async_copy(entp_hbm.at[pl.ds(hvec[k] * PKD, PKD)],
                                 hbuf.at[pl.ds(b * PKD, PKD)], sem)
                pltpu.async_copy(entp_hbm.at[pl.ds(tvec[k] * PKD, PKD)],
                                 tbuf.at[pl.ds(b * PKD, PKD)], sem)
            return carry

        lax.fori_loop(0, GPC, fire_g, 0)

    def unpack(w):
        lo = plsc.bitcast(lax.shift_left(w, 16), jnp.float32)
        hi = plsc.bitcast(lax.bitwise_and(w, himask), jnp.float32)
        return lo, hi

    def drain_and_compute(c):
        hbuf, tbuf, sem = bufs[c % 2]
        pltpu.make_async_copy(entp_hbm.at[pl.ds(0, CHUNK * PKD)], hbuf,
                              sem).wait()
        pltpu.make_async_copy(entp_hbm.at[pl.ds(0, CHUNK * PKD)], tbuf,
                              sem).wait()

        def group(g, carry):
            rowbase = (g * 16 + lane) * PKD
            rids = r_sm[pl.ds(c * CHUNK + g * 16, 16)] * DIM
            acc = jnp.zeros((16,), jnp.float32)
            for j in range(PKD):
                hw = plsc.load_gather(hbuf, [rowbase + j])
                tw = plsc.load_gather(tbuf, [rowbase + j])
                h_lo, h_hi = unpack(hw)
                t_lo, t_hi = unpack(tw)
                r_lo = plsc.load_gather(rel_v, [rids + 2 * j])
                r_hi = plsc.load_gather(rel_v, [rids + 2 * j + 1])
                dv_lo = (h_lo + r_lo) - t_lo
                dv_hi = (h_hi + r_hi) - t_hi
                acc = acc + dv_lo * dv_lo + dv_hi * dv_hi
            yi = jnp.int32(0x5F3759DF) - lax.shift_right_logical(
                plsc.bitcast(acc, jnp.int32), 1)
            y = plsc.bitcast(yi, jnp.float32)
            for _ in range(3):
                y = y * (1.5 - 0.5 * acc * y * y)
            scores[pl.ds(c * CHUNK + g * 16, 16)] = acc * y
            return carry

        lax.fori_loop(0, GPC, group, 0)

    fire(0)
    for c in range(NCHUNK):
        if c + 1 < NCHUNK:
            fire(c + 1)
        drain_and_compute(c)

    pltpu.sync_copy(scores, out_hbm.at[pl.ds(wid * BPW, BPW)])


@functools.partial(jax.jit, static_argnames=())
def kernel(h_ids, r_typ, t_ids, ent_emb, rel_emb):
    h2 = h_ids.astype(jnp.int32)
    r2 = r_typ.astype(jnp.int32)
    t2 = t_ids.astype(jnp.int32)
    entp = lax.bitcast_convert_type(
        ent_emb.astype(jnp.bfloat16).reshape(ENT_N * PKD, 2), jnp.int32)
    rel1 = rel_emb.reshape(REL_N * DIM)

    mesh = plsc.VectorSubcoreMesh(core_axis_name="c", subcore_axis_name="s")
    run = pl.kernel(
        _tec_body,
        out_type=jax.ShapeDtypeStruct((B,), jnp.float32),
        mesh=mesh,
        compiler_params=pltpu.CompilerParams(needs_layout_passes=False),
        scratch_types=[
            pltpu.VMEM((BPW,), jnp.int32),
            pltpu.VMEM((BPW,), jnp.int32),
            pltpu.VMEM((BPW,), jnp.int32),
            pltpu.VMEM((REL_N * DIM,), jnp.float32),
            pltpu.VMEM((CHUNK * PKD,), jnp.int32),
            pltpu.VMEM((CHUNK * PKD,), jnp.int32),
            pltpu.VMEM((CHUNK * PKD,), jnp.int32),
            pltpu.VMEM((CHUNK * PKD,), jnp.int32),
            pltpu.VMEM((BPW,), jnp.float32),
            pltpu.SemaphoreType.DMA,
            pltpu.SemaphoreType.DMA,
        ],
    )
    return run(h2, r2, t2, entp, rel1)


# speedup vs baseline: 23.1167x; 23.1167x over previous
import functools

import jax
import jax.numpy as jnp
from jax import lax
from jax.experimental import pallas as pl
from jax.experimental.pallas import tpu as pltpu
from jax.experimental.pallas import tpu_sc as plsc

ENT_N = 1000000
REL_N = 1000
DIM = 32
B = 16384

NC = 2
NS = 16
NW = NC * NS
BPW = B // NW
CHUNK = 128
NCHUNK = BPW // CHUNK
GPC = CHUNK // 16


def _tec_body(h_hbm, r_hbm, t_hbm, ent_hbm, rel1_hbm, out_hbm,
              h_sm, r_sm, t_sm, rel_v,
              h_c0, t_c0, h_c1, t_c1, scores, sem0, sem1):
    wid = lax.axis_index("s") * NC + lax.axis_index("c")

    pltpu.sync_copy(h_hbm.at[pl.ds(wid * BPW, BPW)], h_sm)
    pltpu.sync_copy(r_hbm.at[pl.ds(wid * BPW, BPW)], r_sm)
    pltpu.sync_copy(t_hbm.at[pl.ds(wid * BPW, BPW)], t_sm)
    pltpu.sync_copy(rel1_hbm, rel_v)

    bufs = [(h_c0, t_c0, sem0), (h_c1, t_c1, sem1)]
    lane = lax.iota(jnp.int32, 16)

    def fire(c):
        h_rows, t_rows, sem = bufs[c % 2]

        def fire_g(g, carry):
            base = c * CHUNK + g * 16
            hvec = h_sm[pl.ds(base, 16)]
            tvec = t_sm[pl.ds(base, 16)]
            for k in range(16):
                b = g * 16 + k
                pltpu.async_copy(ent_hbm.at[pl.ds(hvec[k], 1)],
                                 h_rows.at[pl.ds(b, 1)], sem)
                pltpu.async_copy(ent_hbm.at[pl.ds(tvec[k], 1)],
                                 t_rows.at[pl.ds(b, 1)], sem)
            return carry

        lax.fori_loop(0, GPC, fire_g, 0)

    def drain_and_compute(c):
        h_rows, t_rows, sem = bufs[c % 2]
        pltpu.make_async_copy(ent_hbm.at[pl.ds(0, CHUNK)], h_rows, sem).wait()
        pltpu.make_async_copy(ent_hbm.at[pl.ds(0, CHUNK)], t_rows, sem).wait()

        def group(g, carry):
            row = g * 16 + lane
            rids = r_sm[pl.ds(c * CHUNK + g * 16, 16)] * DIM
            acc = jnp.zeros((16,), jnp.float32)
            for d in range(DIM):
                col = jnp.full((16,), d, jnp.int32)
                hv = plsc.load_gather(h_rows, [row, col])
                tv = plsc.load_gather(t_rows, [row, col])
                rv = plsc.load_gather(rel_v, [rids + d])
                dv = (hv + rv) - tv
                acc = acc + dv * dv
            yi = jnp.int32(0x5F3759DF) - lax.shift_right_logical(
                plsc.bitcast(acc, jnp.int32), 1)
            y = plsc.bitcast(yi, jnp.float32)
            for _ in range(3):
                y = y * (1.5 - 0.5 * acc * y * y)
            scores[pl.ds(c * CHUNK + g * 16, 16)] = acc * y
            return carry

        lax.fori_loop(0, GPC, group, 0)

    fire(0)
    for c in range(NCHUNK):
        if c + 1 < NCHUNK:
            fire(c + 1)
        drain_and_compute(c)

    pltpu.sync_copy(scores, out_hbm.at[pl.ds(wid * BPW, BPW)])


@functools.partial(jax.jit, static_argnames=())
def kernel(h_ids, r_typ, t_ids, ent_emb, rel_emb):
    h2 = h_ids.astype(jnp.int32)
    r2 = r_typ.astype(jnp.int32)
    t2 = t_ids.astype(jnp.int32)
    rel1 = rel_emb.reshape(REL_N * DIM)

    mesh = plsc.VectorSubcoreMesh(core_axis_name="c", subcore_axis_name="s")
    run = pl.kernel(
        _tec_body,
        out_type=jax.ShapeDtypeStruct((B,), jnp.float32),
        mesh=mesh,
        compiler_params=pltpu.CompilerParams(needs_layout_passes=False),
        scratch_types=[
            pltpu.VMEM((BPW,), jnp.int32),
            pltpu.VMEM((BPW,), jnp.int32),
            pltpu.VMEM((BPW,), jnp.int32),
            pltpu.VMEM((REL_N * DIM,), jnp.float32),
            pltpu.VMEM((CHUNK, DIM), jnp.float32),
            pltpu.VMEM((CHUNK, DIM), jnp.float32),
            pltpu.VMEM((CHUNK, DIM), jnp.float32),
            pltpu.VMEM((CHUNK, DIM), jnp.float32),
            pltpu.VMEM((BPW,), jnp.float32),
            pltpu.SemaphoreType.DMA,
            pltpu.SemaphoreType.DMA,
        ],
    )
    return run(h2, r2, t2, ent_emb, rel1)
